# baseline (speedup 1.0000x reference)
import functools

import jax
import jax.numpy as jnp
from jax import lax
from jax.experimental import pallas as pl
from jax.experimental.pallas import tpu as pltpu
from jax.experimental.pallas import tpu_sc as plsc

B = 4096
P = 20
T = 26
V = 100000
D = 64
IN = (T + 1) * D

NC = 2
NS = 16
NW = NC * NS
BPW = B // NW
SUB = 32
NSUB = BPW // SUB
ROWS = SUB * P
IDXW = 128
NG = ROWS // IDXW


NGW = BPW * P // IDXW


SPT = 13
SAMP = B // NS
NGRP = SAMP // 16
DBLK = 4
NSUBB = D // DBLK
NBLK = SPT * NSUBB
SEG = DBLK * V
APT = P * 64
HWORDS = (NGRP // 2) * APT


def _sc2_body(idx_hbm, emb_hbm, out_hbm, iv, addr_v, dst0, dst1, ost,
              sp0, sp1, ssem0, ssem1, gsem0, gsem1):
    c = lax.axis_index("c")
    s = lax.axis_index("s")
    t0 = c * SPT
    stager = s == 0
    sps = (sp0, sp1)
    ssems = (ssem0, ssem1)
    dsts = (dst0, dst1)

    def stage(k, par):
        tl = k // NSUBB
        r = k % NSUBB
        base = ((t0 + tl) * D + r * DBLK) * V
        pltpu.async_copy(emb_hbm.at[pl.ds(base, SEG)], sps[par], ssems[par])

    @pl.when(stager)
    def _():
        stage(0, 0)
        stage(1, 1)

    def addr_gen(t):
        pltpu.sync_copy(idx_hbm.at[t, :, pl.ds(s * SAMP, SAMP)], iv)

        def pbody(p, carry):
            def gbody(g, carry2):
                lanes = lax.iota(jnp.int32, 16)
                soff = (lanes % DBLK) * V
                v16 = iv[p, pl.ds(g * 16, 16)]
                for qr in range(4):
                    perm = lanes // DBLK + 4 * qr
                    ex = lax.gather(
                        v16, perm[:, None],
                        lax.GatherDimensionNumbers(
                            offset_dims=(), collapsed_slice_dims=(0,),
                            start_index_map=(0,)),
                        (1,), mode=lax.GatherScatterMode.PROMISE_IN_BOUNDS)
                    addr_v[pl.ds(g * APT + p * 64 + qr * 16, 16)] = ex + soff
                return carry2
            lax.fori_loop(0, NGRP, gbody, 0)
            return carry
        lax.fori_loop(0, P, pbody, 0)

    def acc_half(half, dstref, r):
        def gl_body(gl, carry):
            lanes = lax.iota(jnp.int32, 16)
            for qr in range(4):
                o0 = gl * APT + qr * 16
                acc = dstref[pl.ds(o0, 16)]
                for p in range(1, P):
                    acc = acc + dstref[pl.ds(o0 + p * 64, 16)]
                jq = (half * (NGRP // 2) + gl) * 4 + qr
                rowc = 4 * jq + lanes // DBLK
                colc = r * DBLK + lanes % DBLK
                plsc.store_scatter(ost, [rowc, colc], acc)
            return carry
        lax.fori_loop(0, NGRP // 2, gl_body, 0)

    def outer(k2, carry):
        for b in range(2):
            k = 2 * k2 + b
            tl = k // NSUBB
            r = k % NSUBB
            t = t0 + tl

            if b == 0:
                @pl.when(r == 0)
                def _():
                    addr_gen(t)

            @pl.when(stager)
            def _():
                pltpu.make_async_copy(
                    emb_hbm.at[pl.ds(0, SEG)], sps[b], ssems[b]
                ).wait()
            plsc.subcore_barrier()

            cp0 = pltpu.async_copy(
                sps[b].at[addr_v.at[pl.ds(0, HWORDS)]], dsts[0], gsem0)
            cp1 = pltpu.async_copy(
                sps[b].at[addr_v.at[pl.ds(HWORDS, HWORDS)]], dsts[1], gsem1)
            cp0.wait()
            acc_half(0, dsts[0], r)
            cp1.wait()
            acc_half(1, dsts[1], r)

            plsc.subcore_barrier()

            @pl.when(stager & (k + 2 < NBLK))
            def _():
                stage(k + 2, b)

            if b == 1:
                @pl.when(r == NSUBB - 1)
                def _():
                    pltpu.sync_copy(
                        ost,
                        out_hbm.at[pl.ds(s * SAMP, SAMP), pl.ds(t * D, D)],
                    )
        return carry

    lax.fori_loop(0, NBLK // 2, outer, 0)


def _sc_embed2(idxp, embf):
    mesh = plsc.VectorSubcoreMesh(
        core_axis_name="c", subcore_axis_name="s", num_cores=NC, num_subcores=NS
    )
    return pl.kernel(
        _sc2_body,
        out_type=jax.ShapeDtypeStruct((B, T * D), jnp.float32),
        mesh=mesh,
        scratch_types=[
            pltpu.VMEM((P, SAMP), jnp.int32),
            pltpu.VMEM((2 * HWORDS,), jnp.int32),
            pltpu.VMEM((HWORDS,), jnp.float32),
            pltpu.VMEM((HWORDS,), jnp.float32),
            pltpu.VMEM((SAMP, D), jnp.float32),
            pltpu.VMEM_SHARED((SEG,), jnp.float32),
            pltpu.VMEM_SHARED((SEG,), jnp.float32),
            pltpu.SemaphoreType.DMA,
            pltpu.SemaphoreType.DMA,
            pltpu.SemaphoreType.DMA,
            pltpu.SemaphoreType.DMA,
        ],
        compiler_params=pltpu.CompilerParams(
            use_tc_tiling_on_sc=False, needs_layout_passes=False
        ),
        name="sc_embed_cols",
    )(idxp, embf)


NSTEP = T * NSUB


def _sc_embed_body(idx_hbm, emb_hbm, out_hbm, gidx_v, rows_v, out_v, sem0, sem1):
    wid = lax.axis_index("s") * NC + lax.axis_index("c")
    b0 = wid * BPW
    sems = (sem0, sem1)

    def stage_issue(kn, par):
        pair = kn // 8
        c = (kn // 2) % 4
        h = kn % 2
        t = 2 * pair + h
        pltpu.sync_copy(idx_hbm.at[t, wid, pl.ds(c * NG, NG)], gidx_v.at[par])
        for j in range(NG):
            pltpu.async_copy(
                emb_hbm.at[t].at[gidx_v.at[par, j]],
                rows_v.at[par, pl.ds(j * IDXW, IDXW)],
                sems[par],
            )

    stage_issue(0, 0)

    def outer(k2, carry):
        for b in range(2):
            k = 2 * k2 + b
            kn = k + 1

            @pl.when(kn < NSTEP)
            def _():
                stage_issue(kn, 1 - b)

            pltpu.make_async_copy(
                emb_hbm.at[0, pl.ds(0, ROWS), :], rows_v.at[b], sems[b]
            ).wait()

            def acc_body(s, carry3):
                r0 = s * P
                for q in range(D // 16):
                    sl = pl.ds(q * 16, 16)
                    acc = rows_v[b, r0, sl]
                    for p in range(1, P):
                        acc = acc + rows_v[b, r0 + p, sl]
                    out_v[s, pl.ds(b * D + q * 16, 16)] = acc
                return carry3

            lax.fori_loop(0, SUB, acc_body, 0)
            if b == 1:
                pair = k // 8
                c = (k // 2) % 4
                pltpu.sync_copy(
                    out_v,
                    out_hbm.at[pl.ds(b0 + c * SUB, SUB), pl.ds(pair * 2 * D, 2 * D)],
                )
        return carry

    lax.fori_loop(0, NSTEP // 2, outer, 0)


def _sc_embed(idx4d, emb3d):
    mesh = plsc.VectorSubcoreMesh(
        core_axis_name="c", subcore_axis_name="s", num_cores=NC, num_subcores=NS
    )
    return pl.kernel(
        _sc_embed_body,
        out_type=jax.ShapeDtypeStruct((B, T * D), jnp.float32),
        mesh=mesh,
        scratch_types=[
            pltpu.VMEM((2, NG, IDXW), jnp.int32),
            pltpu.VMEM((2, ROWS, D), jnp.float32),
            pltpu.VMEM((SUB, 2 * D), jnp.float32),
            pltpu.SemaphoreType.DMA,
            pltpu.SemaphoreType.DMA,
        ],
        compiler_params=pltpu.CompilerParams(use_tc_tiling_on_sc=False),
        name="sc_embed_bag",
    )(idx4d, emb3d)


BM = 256


def _dense_body(dx_ref, sp_ref, bw0, bb0, bw1, bb1, bw2, bb2,
                vt, wt, db, tw0, tb0, tw1, tb1, tw2, tb2, out_ref):
    f32 = jnp.float32
    x = dx_ref[...]
    x = jnp.maximum(jnp.dot(x, bw0[...], preferred_element_type=f32) + bb0[...], 0.0)
    x = jnp.maximum(jnp.dot(x, bw1[...], preferred_element_type=f32) + bb1[...], 0.0)
    x = jnp.maximum(jnp.dot(x, bw2[...], preferred_element_type=f32) + bb2[...], 0.0)
    comb = jnp.concatenate([sp_ref[...], x], axis=1)
    xl = comb
    for l in range(3):
        xv = jnp.dot(xl, vt[l], preferred_element_type=f32)
        xw = jnp.dot(xv, wt[l], preferred_element_type=f32)
        xl = comb * (xw + db[l]) + xl
    y = jnp.maximum(jnp.dot(xl, tw0[...], preferred_element_type=f32) + tb0[...], 0.0)
    y = jnp.maximum(jnp.dot(y, tw1[...], preferred_element_type=f32) + tb1[...], 0.0)
    out_ref[...] = jnp.sum(y * tw2[...], axis=1, keepdims=True) + tb2[...]


def _full(shape):
    return pl.BlockSpec(shape, lambda i: (0,) * len(shape))


def _dense(dx_p, sparse, bw0t, bb0, bw1t, bb1, bw2t, bb2,
           vt, wt, db, tw0t, tb0, tw1t, tb1, tw2r, tb2):
    grid = (B // BM,)
    return pl.pallas_call(
        _dense_body,
        grid=grid,
        in_specs=[
            pl.BlockSpec((BM, 128), lambda i: (i, 0)),
            pl.BlockSpec((BM, T * D), lambda i: (i, 0)),
            _full(bw0t.shape), _full(bb0.shape),
            _full(bw1t.shape), _full(bb1.shape),
            _full(bw2t.shape), _full(bb2.shape),
            _full(vt.shape), _full(wt.shape), _full(db.shape),
            _full(tw0t.shape), _full(tb0.shape),
            _full(tw1t.shape), _full(tb1.shape),
            _full(tw2r.shape), _full(tb2.shape),
        ],
        out_specs=pl.BlockSpec((BM, 1), lambda i: (i, 0)),
        out_shape=jax.ShapeDtypeStruct((B, 1), jnp.float32),
        compiler_params=pltpu.CompilerParams(
            dimension_semantics=("arbitrary",),
        ),
    )(dx_p, sparse, bw0t, bb0, bw1t, bb1, bw2t, bb2,
      vt, wt, db, tw0t, tb0, tw1t, tb1, tw2r, tb2)


def kernel(dense_x, lS_o, lS_i, emb, bw0, bb0, bw1, bb1, bw2, bb2,
           tw0, tb0, tw1, tb1, tw2, tb2, dcn_W, dcn_V, dcn_b):
    del lS_o
    idxp = lS_i.reshape(T, B, P).transpose(0, 2, 1)
    embf = jnp.transpose(emb, (0, 2, 1)).reshape(-1)
    dx_p = jnp.zeros((B, 128), jnp.float32).at[:, :13].set(dense_x)
    bw0t = jnp.zeros((128, bw0.shape[0]), jnp.float32).at[:13, :].set(bw0.T)
    vt = jnp.roll(jnp.transpose(dcn_V, (0, 2, 1)), -D, axis=1)
    wt = jnp.roll(jnp.transpose(dcn_W, (0, 2, 1)), -D, axis=2)
    db = jnp.roll(dcn_b, -D, axis=1).reshape(3, 1, IN)
    tw0t = jnp.roll(tw0.T, -D, axis=0)
    tw1t = tw1.T
    tw2r = tw2.reshape(1, -1)
    bb0r = bb0.reshape(1, -1)
    bb1r = bb1.reshape(1, -1)
    bb2r = bb2.reshape(1, -1)
    tb0r = tb0.reshape(1, -1)
    tb1r = tb1.reshape(1, -1)
    tb2r = tb2.reshape(1, 1)

    sparse = _sc_embed2(idxp, embf)
    return _dense(dx_p, sparse, bw0t, bb0r, bw1.T, bb1r, bw2.T, bb2r,
                  vt, wt, db, tw0t, tb0r, tw1t, tb1r, tw2r, tb2r)


# speedup vs baseline: 1.2936x; 1.2936x over previous
import functools

import jax
import jax.numpy as jnp
from jax import lax
from jax.experimental import pallas as pl
from jax.experimental.pallas import tpu as pltpu
from jax.experimental.pallas import tpu_sc as plsc

B = 4096
P = 20
T = 26
V = 100000
D = 64
IN = (T + 1) * D

NC = 2
NS = 16
NW = NC * NS
BPW = B // NW
SUB = 32
NSUB = BPW // SUB
ROWS = SUB * P
IDXW = 128
NG = ROWS // IDXW


NGW = BPW * P // IDXW


def _sc_embed_body(nt, idx_hbm, emb_hbm, out_hbm, gidx_v, rows_v, out_v, sem0, sem1):
    wid = lax.axis_index("s") * NC + lax.axis_index("c")
    b0 = wid * BPW
    sems = (sem0, sem1)
    nstep = nt * NSUB

    def stage_issue(kn, par):
        pair = kn // 8
        c = (kn // 2) % 4
        h = kn % 2
        t = 2 * pair + h
        pltpu.sync_copy(idx_hbm.at[t, wid, pl.ds(c * NG, NG)], gidx_v.at[par])
        for j in range(NG):
            pltpu.async_copy(
                emb_hbm.at[t].at[gidx_v.at[par, j]],
                rows_v.at[par, pl.ds(j * IDXW, IDXW)],
                sems[par],
            )

    stage_issue(0, 0)

    def outer(k2, carry):
        for b in range(2):
            k = 2 * k2 + b
            kn = k + 1

            @pl.when(kn < nstep)
            def _():
                stage_issue(kn, 1 - b)

            pltpu.make_async_copy(
                emb_hbm.at[0, pl.ds(0, ROWS), :], rows_v.at[b], sems[b]
            ).wait()

            def acc_body(s, carry3):
                r0 = s * P
                for q in range(D // 16):
                    sl = pl.ds(q * 16, 16)
                    acc = rows_v[b, r0, sl]
                    for p in range(1, P):
                        acc = acc + rows_v[b, r0 + p, sl]
                    out_v[s, pl.ds(b * D + q * 16, 16)] = acc
                return carry3

            lax.fori_loop(0, SUB, acc_body, 0)
            if b == 1:
                pair = k // 8
                c = (k // 2) % 4
                pltpu.sync_copy(
                    out_v,
                    out_hbm.at[pl.ds(b0 + c * SUB, SUB), pl.ds(pair * 2 * D, 2 * D)],
                )
        return carry

    lax.fori_loop(0, nstep // 2, outer, 0)


def _sc_embed(idx4d, emb3d):
    import functools as _ft
    nt = emb3d.shape[0]
    mesh = plsc.VectorSubcoreMesh(
        core_axis_name="c", subcore_axis_name="s", num_cores=NC, num_subcores=NS
    )
    return pl.kernel(
        _ft.partial(_sc_embed_body, nt),
        out_type=jax.ShapeDtypeStruct((B, nt * D), jnp.float32),
        mesh=mesh,
        scratch_types=[
            pltpu.VMEM((2, NG, IDXW), jnp.int32),
            pltpu.VMEM((2, ROWS, D), jnp.float32),
            pltpu.VMEM((SUB, 2 * D), jnp.float32),
            pltpu.SemaphoreType.DMA,
            pltpu.SemaphoreType.DMA,
        ],
        compiler_params=pltpu.CompilerParams(use_tc_tiling_on_sc=False),
        name="sc_embed_bag",
    )(idx4d, emb3d)


BM = 256


def _dense_body(dx_ref, sp1_ref, sp2_ref, bw0, bb0, bw1, bb1, bw2, bb2,
                vt, wt, db, tw0, tb0, tw1, tb1, tw2, tb2, out_ref):
    f32 = jnp.float32
    x = dx_ref[...]
    x = jnp.maximum(jnp.dot(x, bw0[...], preferred_element_type=f32) + bb0[...], 0.0)
    x = jnp.maximum(jnp.dot(x, bw1[...], preferred_element_type=f32) + bb1[...], 0.0)
    x = jnp.maximum(jnp.dot(x, bw2[...], preferred_element_type=f32) + bb2[...], 0.0)
    comb = jnp.concatenate([sp1_ref[...], sp2_ref[...], x], axis=1)
    xl = comb
    for l in range(3):
        xv = jnp.dot(xl, vt[l], preferred_element_type=f32)
        xw = jnp.dot(xv, wt[l], preferred_element_type=f32)
        xl = comb * (xw + db[l]) + xl
    y = jnp.maximum(jnp.dot(xl, tw0[...], preferred_element_type=f32) + tb0[...], 0.0)
    y = jnp.maximum(jnp.dot(y, tw1[...], preferred_element_type=f32) + tb1[...], 0.0)
    out_ref[...] = jnp.sum(y * tw2[...], axis=1, keepdims=True) + tb2[...]


def _full(shape):
    return pl.BlockSpec(shape, lambda i: (0,) * len(shape))


def _dense(dx_p, sp1, sp2, bw0t, bb0, bw1t, bb1, bw2t, bb2,
           vt, wt, db, tw0t, tb0, tw1t, tb1, tw2r, tb2):
    grid = (B // BM,)
    return pl.pallas_call(
        _dense_body,
        grid=grid,
        in_specs=[
            pl.BlockSpec((BM, 128), lambda i: (i, 0)),
            pl.BlockSpec((BM, sp1.shape[1]), lambda i: (i, 0)),
            pl.BlockSpec((BM, sp2.shape[1]), lambda i: (i, 0)),
            _full(bw0t.shape), _full(bb0.shape),
            _full(bw1t.shape), _full(bb1.shape),
            _full(bw2t.shape), _full(bb2.shape),
            _full(vt.shape), _full(wt.shape), _full(db.shape),
            _full(tw0t.shape), _full(tb0.shape),
            _full(tw1t.shape), _full(tb1.shape),
            _full(tw2r.shape), _full(tb2.shape),
        ],
        out_specs=pl.BlockSpec((BM, 1), lambda i: (i, 0)),
        out_shape=jax.ShapeDtypeStruct((B, 1), jnp.float32),
        compiler_params=pltpu.CompilerParams(
            dimension_semantics=("arbitrary",),
        ),
    )(dx_p, sp1, sp2, bw0t, bb0, bw1t, bb1, bw2t, bb2,
      vt, wt, db, tw0t, tb0, tw1t, tb1, tw2r, tb2)


def kernel(dense_x, lS_o, lS_i, emb, bw0, bb0, bw1, bb1, bw2, bb2,
           tw0, tb0, tw1, tb1, tw2, tb2, dcn_W, dcn_V, dcn_b):
    del lS_o
    idx4d = lS_i.reshape(T, NW, NGW, IDXW)
    dx_p = jnp.zeros((B, 128), jnp.float32).at[:, :13].set(dense_x)
    bw0t = jnp.zeros((128, bw0.shape[0]), jnp.float32).at[:13, :].set(bw0.T)
    vt = jnp.roll(jnp.transpose(dcn_V, (0, 2, 1)), -D, axis=1)
    wt = jnp.roll(jnp.transpose(dcn_W, (0, 2, 1)), -D, axis=2)
    db = jnp.roll(dcn_b, -D, axis=1).reshape(3, 1, IN)
    tw0t = jnp.roll(tw0.T, -D, axis=0)
    tw1t = tw1.T
    tw2r = tw2.reshape(1, -1)
    bb0r = bb0.reshape(1, -1)
    bb1r = bb1.reshape(1, -1)
    bb2r = bb2.reshape(1, -1)
    tb0r = tb0.reshape(1, -1)
    tb1r = tb1.reshape(1, -1)
    tb2r = tb2.reshape(1, 1)

    SPLIT = 14
    sp1 = _sc_embed(idx4d[:SPLIT], emb[:SPLIT])
    sp2 = _sc_embed(idx4d[SPLIT:], emb[SPLIT:])
    return _dense(dx_p, sp1, sp2, bw0t, bb0r, bw1.T, bb1r, bw2.T, bb2r,
                  vt, wt, db, tw0t, tb0r, tw1t, tb1r, tw2r, tb2r)


# speedup vs baseline: 1.3998x; 1.0821x over previous
import functools

import jax
import jax.numpy as jnp
from jax import lax
from jax.experimental import pallas as pl
from jax.experimental.pallas import tpu as pltpu
from jax.experimental.pallas import tpu_sc as plsc

B = 4096
P = 20
T = 26
V = 100000
D = 64
IN = (T + 1) * D

NC = 2
NS = 16
NW = NC * NS
BPW = B // NW
SUB = 32
NSUB = BPW // SUB
ROWS = SUB * P
IDXW = 128
NG = ROWS // IDXW


NGW = BPW * P // IDXW


NSTEP = T * NSUB


def _sc_embed_body(idx_hbm, emb_hbm, out_hbm, gidx_v, rows_v, out_v, sem0, sem1, osem):
    wid = lax.axis_index("s") * NC + lax.axis_index("c")
    b0 = wid * BPW
    sems = (sem0, sem1)

    def stage_issue(kn, par):
        pair = kn // 8
        c = (kn // 2) % 4
        h = kn % 2
        t = 2 * pair + h
        pltpu.sync_copy(idx_hbm.at[t, wid, pl.ds(c * NG, NG)], gidx_v.at[par])
        for j in range(NG):
            pltpu.async_copy(
                emb_hbm.at[t].at[gidx_v.at[par, j]],
                rows_v.at[par, pl.ds(j * IDXW, IDXW)],
                sems[par],
            )

    stage_issue(0, 0)

    def outer(k2, carry):
        for b in range(2):
            k = 2 * k2 + b
            kn = k + 1

            @pl.when(kn < NSTEP)
            def _():
                stage_issue(kn, 1 - b)

            pltpu.make_async_copy(
                emb_hbm.at[0, pl.ds(0, ROWS), :], rows_v.at[b], sems[b]
            ).wait()

            ov = k2 % 2
            if b == 0:
                @pl.when(k2 >= 2)
                def _():
                    pltpu.make_async_copy(
                        out_v.at[ov],
                        out_hbm.at[pl.ds(0, SUB), pl.ds(0, 2 * D)],
                        osem,
                    ).wait()

            def acc_body(s, carry3):
                for u in range(2):
                    r0 = (2 * s + u) * P
                    for q in range(D // 16):
                        sl = pl.ds(q * 16, 16)
                        acc = rows_v[b, r0, sl]
                        for p in range(1, P):
                            acc = acc + rows_v[b, r0 + p, sl]
                        out_v[ov, 2 * s + u, pl.ds(b * D + q * 16, 16)] = acc
                return carry3

            lax.fori_loop(0, SUB // 2, acc_body, 0)
            if b == 1:
                pair = k // 8
                c = (k // 2) % 4
                pltpu.async_copy(
                    out_v.at[ov],
                    out_hbm.at[pl.ds(b0 + c * SUB, SUB), pl.ds(pair * 2 * D, 2 * D)],
                    osem,
                )
        return carry

    lax.fori_loop(0, NSTEP // 2, outer, 0)
    for _ in range(2):
        pltpu.make_async_copy(
            out_v.at[0],
            out_hbm.at[pl.ds(0, SUB), pl.ds(0, 2 * D)],
            osem,
        ).wait()


def _sc_embed(idx4d, emb3d):
    mesh = plsc.VectorSubcoreMesh(
        core_axis_name="c", subcore_axis_name="s", num_cores=NC, num_subcores=NS
    )
    return pl.kernel(
        _sc_embed_body,
        out_type=jax.ShapeDtypeStruct((B, T * D), jnp.float32),
        mesh=mesh,
        scratch_types=[
            pltpu.VMEM((2, NG, IDXW), jnp.int32),
            pltpu.VMEM((2, ROWS, D), jnp.float32),
            pltpu.VMEM((2, SUB, 2 * D), jnp.float32),
            pltpu.SemaphoreType.DMA,
            pltpu.SemaphoreType.DMA,
            pltpu.SemaphoreType.DMA,
        ],
        compiler_params=pltpu.CompilerParams(use_tc_tiling_on_sc=False),
        name="sc_embed_bag",
    )(idx4d, emb3d)


BM = 256


def _dense_body(dx_ref, sp_ref, bw0, bb0, bw1, bb1, bw2, bb2,
                vt, wt, db, tw0, tb0, tw1, tb1, tw2, tb2, out_ref):
    f32 = jnp.float32
    x = dx_ref[...]
    x = jnp.maximum(jnp.dot(x, bw0[...], preferred_element_type=f32) + bb0[...], 0.0)
    x = jnp.maximum(jnp.dot(x, bw1[...], preferred_element_type=f32) + bb1[...], 0.0)
    x = jnp.maximum(jnp.dot(x, bw2[...], preferred_element_type=f32) + bb2[...], 0.0)
    comb = jnp.concatenate([sp_ref[...], x], axis=1)
    xl = comb
    for l in range(3):
        xv = jnp.dot(xl, vt[l], preferred_element_type=f32)
        xw = jnp.dot(xv, wt[l], preferred_element_type=f32)
        xl = comb * (xw + db[l]) + xl
    y = jnp.maximum(jnp.dot(xl, tw0[...], preferred_element_type=f32) + tb0[...], 0.0)
    y = jnp.maximum(jnp.dot(y, tw1[...], preferred_element_type=f32) + tb1[...], 0.0)
    out_ref[...] = jnp.sum(y * tw2[...], axis=1, keepdims=True) + tb2[...]


def _full(shape):
    return pl.BlockSpec(shape, lambda i: (0,) * len(shape))


def _dense(dx_p, sparse, bw0t, bb0, bw1t, bb1, bw2t, bb2,
           vt, wt, db, tw0t, tb0, tw1t, tb1, tw2r, tb2):
    grid = (B // BM,)
    return pl.pallas_call(
        _dense_body,
        grid=grid,
        in_specs=[
            pl.BlockSpec((BM, 128), lambda i: (i, 0)),
            pl.BlockSpec((BM, T * D), lambda i: (i, 0)),
            _full(bw0t.shape), _full(bb0.shape),
            _full(bw1t.shape), _full(bb1.shape),
            _full(bw2t.shape), _full(bb2.shape),
            _full(vt.shape), _full(wt.shape), _full(db.shape),
            _full(tw0t.shape), _full(tb0.shape),
            _full(tw1t.shape), _full(tb1.shape),
            _full(tw2r.shape), _full(tb2.shape),
        ],
        out_specs=pl.BlockSpec((BM, 1), lambda i: (i, 0)),
        out_shape=jax.ShapeDtypeStruct((B, 1), jnp.float32),
        compiler_params=pltpu.CompilerParams(
            dimension_semantics=("arbitrary",),
        ),
    )(dx_p, sparse, bw0t, bb0, bw1t, bb1, bw2t, bb2,
      vt, wt, db, tw0t, tb0, tw1t, tb1, tw2r, tb2)


def kernel(dense_x, lS_o, lS_i, emb, bw0, bb0, bw1, bb1, bw2, bb2,
           tw0, tb0, tw1, tb1, tw2, tb2, dcn_W, dcn_V, dcn_b):
    del lS_o
    idx4d = lS_i.reshape(T, NW, NGW, IDXW)
    dx_p = jnp.zeros((B, 128), jnp.float32).at[:, :13].set(dense_x)
    bw0t = jnp.zeros((128, bw0.shape[0]), jnp.float32).at[:13, :].set(bw0.T)
    vt = jnp.roll(jnp.transpose(dcn_V, (0, 2, 1)), -D, axis=1)
    wt = jnp.roll(jnp.transpose(dcn_W, (0, 2, 1)), -D, axis=2)
    db = jnp.roll(dcn_b, -D, axis=1).reshape(3, 1, IN)
    tw0t = jnp.roll(tw0.T, -D, axis=0)
    tw1t = tw1.T
    tw2r = tw2.reshape(1, -1)
    bb0r = bb0.reshape(1, -1)
    bb1r = bb1.reshape(1, -1)
    bb2r = bb2.reshape(1, -1)
    tb0r = tb0.reshape(1, -1)
    tb1r = tb1.reshape(1, -1)
    tb2r = tb2.reshape(1, 1)

    sparse = _sc_embed(idx4d, emb)
    return _dense(dx_p, sparse, bw0t, bb0r, bw1.T, bb1r, bw2.T, bb2r,
                  vt, wt, db, tw0t, tb0r, tw1t, tb1r, tw2r, tb2r)


# speedup vs baseline: 1.4005x; 1.0005x over previous
import jax
import jax.numpy as jnp
from jax import lax
from jax.experimental import pallas as pl
from jax.experimental.pallas import tpu as pltpu
from jax.experimental.pallas import tpu_sc as plsc

B = 4096
P = 20
T = 26
V = 100000
D = 64
IN = (T + 1) * D

NC = 2
NS = 16
NW = NC * NS
BPW = B // NW
SUB = 32
NSUB = BPW // SUB
ROWS = SUB * P
IDXW = 128
NG = ROWS // IDXW


NGW = BPW * P // IDXW


NSTEP = T * NSUB


def _sc_embed_body(idx_hbm, emb_hbm, out_hbm, gidx_v, rows_v, out_v, sem0, sem1, osem):
    wid = lax.axis_index("s") * NC + lax.axis_index("c")
    b0 = wid * BPW
    sems = (sem0, sem1)

    def stage_issue(kn, par):
        pair = kn // 8
        c = (kn // 2) % 4
        h = kn % 2
        t = 2 * pair + h
        pltpu.sync_copy(idx_hbm.at[t, wid, pl.ds(c * NG, NG)], gidx_v.at[par])
        for j in range(NG):
            pltpu.async_copy(
                emb_hbm.at[t].at[gidx_v.at[par, j]],
                rows_v.at[par, pl.ds(j * IDXW, IDXW)],
                sems[par],
            )

    stage_issue(0, 0)

    def outer(k2, carry):
        for b in range(2):
            k = 2 * k2 + b
            kn = k + 1

            @pl.when(kn < NSTEP)
            def _():
                stage_issue(kn, 1 - b)

            pltpu.make_async_copy(
                emb_hbm.at[0, pl.ds(0, ROWS), :], rows_v.at[b], sems[b]
            ).wait()

            ov = k2 % 2
            if b == 0:
                @pl.when(k2 >= 2)
                def _():
                    pltpu.make_async_copy(
                        out_v.at[ov],
                        out_hbm.at[pl.ds(0, SUB), pl.ds(0, 2 * D)],
                        osem,
                    ).wait()

            def acc_body(s, carry3):
                for u in range(2):
                    r0 = (2 * s + u) * P
                    for q in range(D // 16):
                        sl = pl.ds(q * 16, 16)
                        acc = rows_v[b, r0, sl]
                        for p in range(1, P):
                            acc = acc + rows_v[b, r0 + p, sl]
                        out_v[ov, 2 * s + u, pl.ds(b * D + q * 16, 16)] = acc
                return carry3

            lax.fori_loop(0, SUB // 2, acc_body, 0)
            if b == 1:
                pair = k // 8
                c = (k // 2) % 4
                pltpu.async_copy(
                    out_v.at[ov],
                    out_hbm.at[pl.ds(b0 + c * SUB, SUB), pl.ds(pair * 2 * D, 2 * D)],
                    osem,
                )
        return carry

    lax.fori_loop(0, NSTEP // 2, outer, 0)
    for _ in range(2):
        pltpu.make_async_copy(
            out_v.at[0],
            out_hbm.at[pl.ds(0, SUB), pl.ds(0, 2 * D)],
            osem,
        ).wait()


def _sc_embed(idx4d, emb3d):
    mesh = plsc.VectorSubcoreMesh(
        core_axis_name="c", subcore_axis_name="s", num_cores=NC, num_subcores=NS
    )
    return pl.kernel(
        _sc_embed_body,
        out_type=jax.ShapeDtypeStruct((B, T * D), jnp.float32),
        mesh=mesh,
        scratch_types=[
            pltpu.VMEM((2, NG, IDXW), jnp.int32),
            pltpu.VMEM((2, ROWS, D), jnp.float32),
            pltpu.VMEM((2, SUB, 2 * D), jnp.float32),
            pltpu.SemaphoreType.DMA,
            pltpu.SemaphoreType.DMA,
            pltpu.SemaphoreType.DMA,
        ],
        compiler_params=pltpu.CompilerParams(use_tc_tiling_on_sc=False),
        name="sc_embed_bag",
    )(idx4d, emb3d)


BM = 256


def _dense_body(dx_ref, sp_ref, bw0, bb0, bw1, bb1, bw2, bb2,
                vt, wt, db, tw0, tb0, tw1, tb1, tw2, tb2, out_ref):
    f32 = jnp.float32
    x = dx_ref[...]
    x = jnp.maximum(jnp.dot(x, bw0[...], preferred_element_type=f32) + bb0[...], 0.0)
    x = jnp.maximum(jnp.dot(x, bw1[...], preferred_element_type=f32) + bb1[...], 0.0)
    x = jnp.maximum(jnp.dot(x, bw2[...], preferred_element_type=f32) + bb2[...], 0.0)
    comb = jnp.concatenate([sp_ref[...], x], axis=1)
    xl = comb
    for l in range(3):
        xv = jnp.dot(xl, vt[l], preferred_element_type=f32)
        xw = jnp.dot(xv, wt[l], preferred_element_type=f32)
        xl = comb * (xw + db[l]) + xl
    y = jnp.maximum(jnp.dot(xl, tw0[...], preferred_element_type=f32) + tb0[...], 0.0)
    y = jnp.maximum(jnp.dot(y, tw1[...], preferred_element_type=f32) + tb1[...], 0.0)
    out_ref[...] = jnp.sum(y * tw2[...], axis=1, keepdims=True) + tb2[...]


def _full(shape):
    return pl.BlockSpec(shape, lambda i: (0,) * len(shape))


def _dense(dx_p, sparse, bw0t, bb0, bw1t, bb1, bw2t, bb2,
           vt, wt, db, tw0t, tb0, tw1t, tb1, tw2r, tb2):
    grid = (B // BM,)
    return pl.pallas_call(
        _dense_body,
        grid=grid,
        in_specs=[
            pl.BlockSpec((BM, 128), lambda i: (i, 0)),
            pl.BlockSpec((BM, T * D), lambda i: (i, 0)),
            _full(bw0t.shape), _full(bb0.shape),
            _full(bw1t.shape), _full(bb1.shape),
            _full(bw2t.shape), _full(bb2.shape),
            _full(vt.shape), _full(wt.shape), _full(db.shape),
            _full(tw0t.shape), _full(tb0.shape),
            _full(tw1t.shape), _full(tb1.shape),
            _full(tw2r.shape), _full(tb2.shape),
        ],
        out_specs=pl.BlockSpec((BM, 1), lambda i: (i, 0)),
        out_shape=jax.ShapeDtypeStruct((B, 1), jnp.float32),
        compiler_params=pltpu.CompilerParams(
            dimension_semantics=("arbitrary",),
        ),
    )(dx_p, sparse, bw0t, bb0, bw1t, bb1, bw2t, bb2,
      vt, wt, db, tw0t, tb0, tw1t, tb1, tw2r, tb2)


def kernel(dense_x, lS_o, lS_i, emb, bw0, bb0, bw1, bb1, bw2, bb2,
           tw0, tb0, tw1, tb1, tw2, tb2, dcn_W, dcn_V, dcn_b):
    del lS_o
    idx4d = lS_i.reshape(T, NW, NGW, IDXW)
    dx_p = jnp.zeros((B, 128), jnp.float32).at[:, :13].set(dense_x)
    bw0t = jnp.zeros((128, bw0.shape[0]), jnp.float32).at[:13, :].set(bw0.T)
    vt = jnp.roll(jnp.transpose(dcn_V, (0, 2, 1)), -D, axis=1)
    wt = jnp.roll(jnp.transpose(dcn_W, (0, 2, 1)), -D, axis=2)
    db = jnp.roll(dcn_b, -D, axis=1).reshape(3, 1, IN)
    tw0t = jnp.roll(tw0.T, -D, axis=0)
    tw1t = tw1.T
    tw2r = tw2.reshape(1, -1)
    bb0r = bb0.reshape(1, -1)
    bb1r = bb1.reshape(1, -1)
    bb2r = bb2.reshape(1, -1)
    tb0r = tb0.reshape(1, -1)
    tb1r = tb1.reshape(1, -1)
    tb2r = tb2.reshape(1, 1)

    sparse = _sc_embed(idx4d, emb)
    return _dense(dx_p, sparse, bw0t, bb0r, bw1.T, bb1r, bw2.T, bb2r,
                  vt, wt, db, tw0t, tb0r, tw1t, tb1r, tw2r, tb2r)
